# baseline (device time: 12694 ns/iter reference)
import jax
import jax.numpy as jnp
from jax import lax
from jax.experimental import pallas as pl
from jax.experimental.pallas import tpu as pltpu

N_DEV = 8
N_PEERS = N_DEV - 1
N_HALVES = 2
SEND_ORDER = (6, 2, 5, 7, 3, 1, 4)


def kernel(x, dy, gamma):
    del gamma
    m_per, d = x.shape
    rows_half = m_per // N_HALVES

    def half_partial(x_ref, dy_ref, h):
        xv = x_ref[pl.ds(h * rows_half, rows_half), :]
        dyv = dy_ref[pl.ds(h * rows_half, rows_half), :]
        s1 = jnp.sum(xv, axis=1, keepdims=True)
        s2 = jnp.sum(xv * xv, axis=1, keepdims=True)
        mu = s1 * (1.0 / d)
        var = s2 * (1.0 / d) - mu * mu
        rstd = lax.rsqrt(var + 1e-5)
        dgamma = jnp.sum(dyv * ((xv - mu) * rstd), axis=0)
        dbeta = jnp.sum(dyv, axis=0)
        return jnp.stack([dgamma, dbeta])

    def body(x_ref, dy_ref, out_ref, send_ref, recv_ref, send_sems, recv_sems):
        my_pos = lax.axis_index("i")

        barrier_sem = pltpu.get_barrier_semaphore()
        for k in range(1, N_DEV):
            pl.semaphore_signal(
                barrier_sem, inc=1,
                device_id=((my_pos + k) % N_DEV,),
                device_id_type=pl.DeviceIdType.MESH,
            )

        def broadcast(h):
            rdmas = []
            for k in SEND_ORDER:
                rdma = pltpu.make_async_remote_copy(
                    src_ref=send_ref.at[h],
                    dst_ref=recv_ref.at[h, k - 1],
                    send_sem=send_sems.at[h, k - 1],
                    recv_sem=recv_sems.at[h, k - 1],
                    device_id=((my_pos + k) % N_DEV,),
                    device_id_type=pl.DeviceIdType.MESH,
                )
                rdma.start()
                rdmas.append(rdma)
            return rdmas

        p0 = half_partial(x_ref, dy_ref, 0)
        send_ref[0, :, :] = p0
        pl.semaphore_wait(barrier_sem, N_PEERS)
        rdmas0 = broadcast(0)

        p1 = half_partial(x_ref, dy_ref, 1)
        send_ref[1, :, :] = p1
        rdmas1 = broadcast(1)

        for rdma in rdmas0:
            rdma.wait()
        acc = p0 + p1 + jnp.sum(recv_ref[0], axis=0)
        for rdma in rdmas1:
            rdma.wait()
        out_ref[:, :] = acc + jnp.sum(recv_ref[1], axis=0)

    return pl.pallas_call(
        body,
        out_shape=jax.ShapeDtypeStruct((2, d), jnp.float32),
        in_specs=[
            pl.BlockSpec(memory_space=pltpu.VMEM),
            pl.BlockSpec(memory_space=pltpu.VMEM),
        ],
        out_specs=pl.BlockSpec(memory_space=pltpu.VMEM),
        scratch_shapes=[
            pltpu.VMEM((N_HALVES, 2, d), jnp.float32),
            pltpu.VMEM((N_HALVES, N_PEERS, 2, d), jnp.float32),
            pltpu.SemaphoreType.DMA((N_HALVES, N_PEERS)),
            pltpu.SemaphoreType.DMA((N_HALVES, N_PEERS)),
        ],
        compiler_params=pltpu.CompilerParams(collective_id=0),
    )(x, dy)


# device time: 12627 ns/iter; 1.0053x vs baseline; 1.0053x over previous
import jax
import jax.numpy as jnp
from jax import lax
from jax.experimental import pallas as pl
from jax.experimental.pallas import tpu as pltpu

N_DEV = 8
N_PEERS = N_DEV - 1
N_HALVES = 2
SEND_ORDER = (6, 2, 5, 7, 3, 1, 4)


def kernel(x, dy, gamma):
    del gamma
    m_per, d = x.shape
    rows_half = m_per // N_HALVES

    def half_partial(x_ref, dy_ref, h):
        xv = x_ref[pl.ds(h * rows_half, rows_half), :]
        dyv = dy_ref[pl.ds(h * rows_half, rows_half), :]
        s1 = jnp.sum(xv, axis=1, keepdims=True)
        s2 = jnp.sum(xv * xv, axis=1, keepdims=True)
        mu = s1 * (1.0 / d)
        var = s2 * (1.0 / d) - mu * mu
        rstd = lax.rsqrt(var + 1e-5)
        dgamma = jnp.sum(dyv * ((xv - mu) * rstd), axis=0)
        dbeta = jnp.sum(dyv, axis=0)
        return jnp.stack([dgamma, dbeta])

    def body(x_ref, dy_ref, out_ref, send_ref, recv_ref, send_sems, recv_sems):
        my_pos = lax.axis_index("i")

        barrier_sem = pltpu.get_barrier_semaphore()
        for k in range(1, N_DEV):
            pl.semaphore_signal(
                barrier_sem, inc=1,
                device_id=((my_pos + k) % N_DEV,),
                device_id_type=pl.DeviceIdType.MESH,
            )

        def broadcast(h):
            rdmas = []
            for k in SEND_ORDER:
                rdma = pltpu.make_async_remote_copy(
                    src_ref=send_ref.at[h],
                    dst_ref=recv_ref.at[h, k - 1],
                    send_sem=send_sems.at[h, k - 1],
                    recv_sem=recv_sems.at[h, k - 1],
                    device_id=((my_pos + k) % N_DEV,),
                    device_id_type=pl.DeviceIdType.MESH,
                )
                rdma.start()
                rdmas.append(rdma)
            return rdmas

        p0 = half_partial(x_ref, dy_ref, 0)
        send_ref[0, :, :] = p0
        pl.semaphore_wait(barrier_sem, N_PEERS)
        rdmas0 = broadcast(0)

        p1 = half_partial(x_ref, dy_ref, 1)
        send_ref[1, :, :] = p1
        rdmas1 = broadcast(1)

        for rdma in rdmas0:
            rdma.wait()
        acc = p0 + p1 + jnp.sum(recv_ref[0], axis=0)
        for rdma, k in reversed(list(zip(rdmas1, SEND_ORDER))):
            rdma.wait()
            acc = acc + recv_ref[1, k - 1, :, :]
        out_ref[:, :] = acc

    return pl.pallas_call(
        body,
        out_shape=jax.ShapeDtypeStruct((2, d), jnp.float32),
        in_specs=[
            pl.BlockSpec(memory_space=pltpu.VMEM),
            pl.BlockSpec(memory_space=pltpu.VMEM),
        ],
        out_specs=pl.BlockSpec(memory_space=pltpu.VMEM),
        scratch_shapes=[
            pltpu.VMEM((N_HALVES, 2, d), jnp.float32),
            pltpu.VMEM((N_HALVES, N_PEERS, 2, d), jnp.float32),
            pltpu.SemaphoreType.DMA((N_HALVES, N_PEERS)),
            pltpu.SemaphoreType.DMA((N_HALVES, N_PEERS)),
        ],
        compiler_params=pltpu.CompilerParams(collective_id=0),
    )(x, dy)
